# Initial kernel scaffold; baseline (speedup 1.0000x reference)
import jax
import jax.numpy as jnp
from jax.experimental import pallas as pl


def kernel(x, W_proj, W_gate, ln_gamma, ln_beta, temperature):
    raise NotImplementedError("write your pallas kernel here")


# speedup vs baseline: 2.3764x; 2.3764x over previous
import functools

import jax
import jax.numpy as jnp
from jax.experimental import pallas as pl
from jax.experimental.pallas import tpu as pltpu

K = 8
Z_COEF = 0.005
LB_COEF = 0.005


def _router_kernel(x_ref, wp_ref, wg_ref, g_ref, b_ref, t_ref,
                   rw_ref, disp_ref, loss_ref,
                   acc_load, acc_z,
                   *, nsteps, n_tokens, n_experts):
    i = pl.program_id(0)

    @pl.when(i == 0)
    def _init():
        acc_load[...] = jnp.zeros_like(acc_load)
        acc_z[...] = jnp.zeros_like(acc_z)
        loss_ref[...] = jnp.zeros_like(loss_ref)

    xb = x_ref[...]
    p = jnp.dot(xb, wp_ref[...], preferred_element_type=jnp.float32)
    mu = jnp.mean(p, axis=-1, keepdims=True)
    c = p - mu
    var = jnp.mean(c * c, axis=-1, keepdims=True)
    nrm = c * jax.lax.rsqrt(var + 1e-5) * g_ref[...] + b_ref[...]
    inv_t = 1.0 / t_ref[0, 0]
    logits = jnp.dot(nrm, wg_ref[...], preferred_element_type=jnp.float32) * inv_t
    m = jnp.max(logits, axis=-1, keepdims=True)
    e = jnp.exp(logits - m)
    rw = e / jnp.sum(e, axis=-1, keepdims=True)
    rw_ref[...] = rw

    col = jax.lax.broadcasted_iota(jnp.int32, rw.shape, 1)
    masked = rw
    sel = jnp.zeros(rw.shape, dtype=jnp.bool_)
    for _ in range(K):
        mx = jnp.max(masked, axis=-1, keepdims=True)
        hit = masked == mx
        first = jnp.min(jnp.where(hit, col, n_experts), axis=-1, keepdims=True)
        new = col == first
        sel = jnp.logical_or(sel, new)
        masked = jnp.where(new, -1.0, masked)
    ssum = jnp.sum(jnp.where(sel, rw, 0.0), axis=-1, keepdims=True)
    disp_ref[...] = jnp.where(sel, rw / (ssum + 1e-6), 0.0)

    acc_load[...] += jnp.sum(rw, axis=0, keepdims=True)
    acc_z[...] += jnp.sum(logits * logits, keepdims=True)

    @pl.when(i == nsteps - 1)
    def _finalize():
        actual = acc_load[...] / n_tokens
        ideal = 1.0 / n_experts
        kl = jnp.sum(ideal * (jnp.log(ideal) - jnp.log(actual)),
                     keepdims=True) / n_experts
        z = acc_z[...] / (n_tokens * n_experts)
        loss_ref[...] = Z_COEF * z + LB_COEF * kl


def kernel(x, W_proj, W_gate, ln_gamma, ln_beta, temperature):
    batch, seq, d = x.shape
    h = W_proj.shape[0]
    e = W_gate.shape[0]
    n = batch * seq
    r = 256
    nsteps = n // r

    x2 = x.reshape(n, d)
    wpT = W_proj.T
    wgT = W_gate.T
    g2 = ln_gamma.reshape(1, h)
    b2 = ln_beta.reshape(1, h)
    t2 = (jnp.abs(temperature) + 1e-6).reshape(1, 1)

    body = functools.partial(_router_kernel, nsteps=nsteps, n_tokens=n,
                             n_experts=e)
    rw, disp, loss = pl.pallas_call(
        body,
        grid=(nsteps,),
        in_specs=[
            pl.BlockSpec((r, d), lambda i: (i, 0)),
            pl.BlockSpec((d, h), lambda i: (0, 0)),
            pl.BlockSpec((h, e), lambda i: (0, 0)),
            pl.BlockSpec((1, h), lambda i: (0, 0)),
            pl.BlockSpec((1, h), lambda i: (0, 0)),
            pl.BlockSpec((1, 1), lambda i: (0, 0)),
        ],
        out_specs=[
            pl.BlockSpec((r, e), lambda i: (i, 0)),
            pl.BlockSpec((r, e), lambda i: (i, 0)),
            pl.BlockSpec((1, 1), lambda i: (0, 0)),
        ],
        out_shape=[
            jax.ShapeDtypeStruct((n, e), jnp.float32),
            jax.ShapeDtypeStruct((n, e), jnp.float32),
            jax.ShapeDtypeStruct((1, 1), jnp.float32),
        ],
        scratch_shapes=[
            pltpu.VMEM((1, e), jnp.float32),
            pltpu.VMEM((1, 1), jnp.float32),
        ],
    )(x2, wpT, wgT, g2, b2, t2)

    return rw, disp.reshape(batch, seq, e), loss[0, 0]


# speedup vs baseline: 3.2453x; 1.3656x over previous
import functools

import jax
import jax.numpy as jnp
from jax.experimental import pallas as pl
from jax.experimental.pallas import tpu as pltpu

K = 8
Z_COEF = 0.005
LB_COEF = 0.005


def _router_kernel(x_ref, wp_ref, wg_ref, g_ref, b_ref, t_ref,
                   rw_ref, disp_ref, loss_ref,
                   acc_load, acc_z,
                   *, nsteps, n_tokens, n_experts):
    i = pl.program_id(0)

    @pl.when(i == 0)
    def _init():
        acc_load[...] = jnp.zeros_like(acc_load)
        acc_z[...] = jnp.zeros_like(acc_z)
        loss_ref[...] = jnp.zeros_like(loss_ref)

    xb = x_ref[...]
    p = jnp.dot(xb, wp_ref[...], preferred_element_type=jnp.float32)
    mu = jnp.mean(p, axis=-1, keepdims=True)
    c = p - mu
    var = jnp.mean(c * c, axis=-1, keepdims=True)
    nrm = c * jax.lax.rsqrt(var + 1e-5) * g_ref[...] + b_ref[...]
    inv_t = 1.0 / t_ref[0, 0]
    logits = jnp.dot(nrm, wg_ref[...], preferred_element_type=jnp.float32) * inv_t
    m = jnp.max(logits, axis=-1, keepdims=True)
    e = jnp.exp(logits - m)
    rw = e / jnp.sum(e, axis=-1, keepdims=True)
    rw_ref[...] = rw

    masked = rw
    mx = jnp.zeros((rw.shape[0], 1), dtype=rw.dtype)
    for _ in range(K):
        mx = jnp.max(masked, axis=-1, keepdims=True)
        masked = jnp.where(masked == mx, -1.0, masked)
    sel = rw >= mx
    ssum = jnp.sum(jnp.where(sel, rw, 0.0), axis=-1, keepdims=True)
    disp_ref[...] = jnp.where(sel, rw / (ssum + 1e-6), 0.0)

    acc_load[...] += jnp.sum(rw, axis=0, keepdims=True)
    acc_z[...] += jnp.sum(logits * logits, keepdims=True)

    @pl.when(i == nsteps - 1)
    def _finalize():
        actual = acc_load[...] / n_tokens
        ideal = 1.0 / n_experts
        kl = jnp.sum(ideal * (jnp.log(ideal) - jnp.log(actual)),
                     keepdims=True) / n_experts
        z = acc_z[...] / (n_tokens * n_experts)
        loss_ref[...] = Z_COEF * z + LB_COEF * kl


def kernel(x, W_proj, W_gate, ln_gamma, ln_beta, temperature):
    batch, seq, d = x.shape
    h = W_proj.shape[0]
    e = W_gate.shape[0]
    n = batch * seq
    r = 256
    nsteps = n // r

    x2 = x.reshape(n, d)
    wpT = W_proj.T
    wgT = W_gate.T
    g2 = ln_gamma.reshape(1, h)
    b2 = ln_beta.reshape(1, h)
    t2 = (jnp.abs(temperature) + 1e-6).reshape(1, 1)

    body = functools.partial(_router_kernel, nsteps=nsteps, n_tokens=n,
                             n_experts=e)
    rw, disp, loss = pl.pallas_call(
        body,
        grid=(nsteps,),
        in_specs=[
            pl.BlockSpec((r, d), lambda i: (i, 0)),
            pl.BlockSpec((d, h), lambda i: (0, 0)),
            pl.BlockSpec((h, e), lambda i: (0, 0)),
            pl.BlockSpec((1, h), lambda i: (0, 0)),
            pl.BlockSpec((1, h), lambda i: (0, 0)),
            pl.BlockSpec((1, 1), lambda i: (0, 0)),
        ],
        out_specs=[
            pl.BlockSpec((r, e), lambda i: (i, 0)),
            pl.BlockSpec((r, e), lambda i: (i, 0)),
            pl.BlockSpec((1, 1), lambda i: (0, 0)),
        ],
        out_shape=[
            jax.ShapeDtypeStruct((n, e), jnp.float32),
            jax.ShapeDtypeStruct((n, e), jnp.float32),
            jax.ShapeDtypeStruct((1, 1), jnp.float32),
        ],
        scratch_shapes=[
            pltpu.VMEM((1, e), jnp.float32),
            pltpu.VMEM((1, 1), jnp.float32),
        ],
    )(x2, wpT, wgT, g2, b2, t2)

    return rw, disp.reshape(batch, seq, e), loss[0, 0]


# speedup vs baseline: 4.0613x; 1.2514x over previous
import functools

import jax
import jax.numpy as jnp
from jax.experimental import pallas as pl
from jax.experimental.pallas import tpu as pltpu

K = 8
Z_COEF = 0.005
LB_COEF = 0.005
NEG = -3.0e38


def _router_kernel(x_ref, wp_ref, wgg_ref, c1_ref, c2_ref, t_ref,
                   rw_ref, disp_ref, loss_ref,
                   acc_load, acc_z,
                   *, nsteps, n_tokens, n_experts):
    i = pl.program_id(0)

    @pl.when(i == 0)
    def _init():
        acc_load[...] = jnp.zeros_like(acc_load)
        acc_z[...] = jnp.zeros_like(acc_z)
        loss_ref[...] = jnp.zeros_like(loss_ref)

    xb = x_ref[...]
    p = jnp.dot(xb, wp_ref[...], preferred_element_type=jnp.float32)
    q = jnp.dot(p, wgg_ref[...], preferred_element_type=jnp.float32)
    mu = jnp.mean(p, axis=-1, keepdims=True)
    msq = jnp.mean(p * p, axis=-1, keepdims=True)
    s = jax.lax.rsqrt(msq - mu * mu + 1e-5)
    inv_t = t_ref[0, 0]
    logits = (q * s - (s * mu) * c1_ref[...] + c2_ref[...]) * inv_t

    masked = logits
    mx = jnp.zeros((logits.shape[0], 1), dtype=logits.dtype)
    m0 = jnp.max(logits, axis=-1, keepdims=True)
    masked = jnp.where(masked == m0, NEG, masked)
    for _ in range(K - 1):
        mx = jnp.max(masked, axis=-1, keepdims=True)
        masked = jnp.where(masked == mx, NEG, masked)
    sel = logits >= mx

    e = jnp.exp(logits - m0)
    rw = e / jnp.sum(e, axis=-1, keepdims=True)
    rw_ref[...] = rw
    ssum = jnp.sum(jnp.where(sel, rw, 0.0), axis=-1, keepdims=True)
    disp_ref[...] = jnp.where(sel, rw / (ssum + 1e-6), 0.0)

    acc_load[...] += jnp.sum(rw, axis=0, keepdims=True)
    acc_z[...] += jnp.sum(logits * logits, keepdims=True)

    @pl.when(i == nsteps - 1)
    def _finalize():
        actual = acc_load[...] / n_tokens
        ideal = 1.0 / n_experts
        kl = jnp.sum(ideal * (jnp.log(ideal) - jnp.log(actual)),
                     keepdims=True) / n_experts
        z = acc_z[...] / (n_tokens * n_experts)
        loss_ref[...] = Z_COEF * z + LB_COEF * kl


def kernel(x, W_proj, W_gate, ln_gamma, ln_beta, temperature):
    batch, seq, d = x.shape
    h = W_proj.shape[0]
    e = W_gate.shape[0]
    n = batch * seq
    r = 256
    nsteps = n // r

    x2 = x.reshape(n, d)
    wpT = W_proj.T
    wgg = W_gate.T * ln_gamma[:, None]
    c1 = jnp.sum(wgg, axis=0, keepdims=True)
    c2 = (ln_beta[None, :] @ W_gate.T)
    t2 = (1.0 / (jnp.abs(temperature) + 1e-6)).reshape(1, 1)

    body = functools.partial(_router_kernel, nsteps=nsteps, n_tokens=n,
                             n_experts=e)
    rw, disp, loss = pl.pallas_call(
        body,
        grid=(nsteps,),
        in_specs=[
            pl.BlockSpec((r, d), lambda i: (i, 0)),
            pl.BlockSpec((d, h), lambda i: (0, 0)),
            pl.BlockSpec((h, e), lambda i: (0, 0)),
            pl.BlockSpec((1, e), lambda i: (0, 0)),
            pl.BlockSpec((1, e), lambda i: (0, 0)),
            pl.BlockSpec((1, 1), lambda i: (0, 0)),
        ],
        out_specs=[
            pl.BlockSpec((r, e), lambda i: (i, 0)),
            pl.BlockSpec((r, e), lambda i: (i, 0)),
            pl.BlockSpec((1, 1), lambda i: (0, 0)),
        ],
        out_shape=[
            jax.ShapeDtypeStruct((n, e), jnp.float32),
            jax.ShapeDtypeStruct((n, e), jnp.float32),
            jax.ShapeDtypeStruct((1, 1), jnp.float32),
        ],
        scratch_shapes=[
            pltpu.VMEM((1, e), jnp.float32),
            pltpu.VMEM((1, 1), jnp.float32),
        ],
    )(x2, wpT, wgg, c1, c2, t2)

    return rw, disp.reshape(batch, seq, e), loss[0, 0]
